# Initial kernel scaffold; baseline (speedup 1.0000x reference)
import jax
import jax.numpy as jnp
from jax.experimental import pallas as pl


def kernel(x, params):
    raise NotImplementedError("write your pallas kernel here")


# speedup vs baseline: 3.8925x; 3.8925x over previous
import numpy as np
import jax
import jax.numpy as jnp
from jax.experimental import pallas as pl

FEAT = 512
NUM_CLASSES = 2
K_NBR = 8
BUF = 4096
NG = 8
B = 4
N_INST = 4096
ATT_D = 128
HALF = 256
PREC = jax.lax.Precision.HIGHEST
NEG = -1e30

_PERM = np.random.RandomState(0).permutation(N_INST)


def _lk(x, s):
    return jnp.where(x >= 0, x, s * x)


def _dot(a, b):
    return jax.lax.dot_general(a, b, (((1,), (0,)), ((), ())), precision=PREC,
                               preferred_element_type=jnp.float32)


def _dot_tn(a, b):
    return jax.lax.dot_general(a, b, (((0,), (0,)), ((), ())), precision=PREC,
                               preferred_element_type=jnp.float32)


def _dot_nt(a, b):
    return jax.lax.dot_general(a, b, (((1,), (1,)), ((), ())), precision=PREC,
                               preferred_element_type=jnp.float32)


def _mil_body(xg, dr_w, dr_b, aV_w, aV_b, aU_w, aU_b, aW, aW_b, sub_wT, sub_b,
              pseudo, subs):
    xb = xg[0, 0]
    mid = jnp.maximum(_dot(xb, dr_w[...]) + dr_b[...], 0.0)
    V = jnp.tanh(_dot(mid, aV_w[...]) + aV_b[...])
    U = jax.nn.sigmoid(_dot(mid, aU_w[...]) + aU_b[...])
    A = jnp.sum((V * U) * aW[...], axis=1, keepdims=True) + aW_b[0, 0]
    A = A - jnp.max(A, axis=0, keepdims=True)
    e = jnp.exp(A)
    t = e / jnp.sum(e, axis=0, keepdims=True)
    af = jnp.sum(mid * t, axis=0, keepdims=True)
    pseudo[0, 0, 0, :] = af[0]
    subs[0, 0, 0, :] = jnp.sum(af * sub_wT[...], axis=1) + sub_b[0]


def _mil_call(xg, p):
    fp = jax.ShapeDtypeStruct
    full = lambda shp: pl.BlockSpec(shp, lambda g, b: (0,) * len(shp))
    return pl.pallas_call(
        _mil_body,
        grid=(NG, B),
        in_specs=[
            pl.BlockSpec((1, 1, N_INST // NG, FEAT), lambda g, b: (b, g, 0, 0)),
            full((FEAT, FEAT)), full((1, FEAT)),
            full((FEAT, ATT_D)), full((1, ATT_D)),
            full((FEAT, ATT_D)), full((1, ATT_D)),
            full((1, ATT_D)), full((1, 1)),
            full((NUM_CLASSES, FEAT)), full((1, NUM_CLASSES)),
        ],
        out_specs=[
            pl.BlockSpec((1, 1, 1, FEAT), lambda g, b: (b, g, 0, 0)),
            pl.BlockSpec((1, 1, 1, NUM_CLASSES), lambda g, b: (g, b, 0, 0)),
        ],
        out_shape=[fp((B, NG, 1, FEAT), jnp.float32),
                   fp((NG, B, 1, NUM_CLASSES), jnp.float32)],
    )(xg, p['dr_w'], p['dr_b'].reshape(1, -1),
      p['aV_w'], p['aV_b'].reshape(1, -1),
      p['aU_w'], p['aU_b'].reshape(1, -1),
      p['aW_w'].reshape(1, -1), p['aW_b'].reshape(1, 1),
      p['sub_w'].T, p['sub_b'].reshape(1, -1))


def _slide_body(slide, laV_w, laV_b, laU_w, laU_b, laW, laW_b, mil_wT, mil_b,
                out_mil, logits):
    for b in range(B):
        rows = slide[b]
        V = jnp.tanh(_dot(rows, laV_w[...]) + laV_b[...])
        U = jax.nn.sigmoid(_dot(rows, laU_w[...]) + laU_b[...])
        A = jnp.sum((V * U) * laW[...], axis=1, keepdims=True) + laW_b[0, 0]
        A = A - jnp.max(A, axis=0, keepdims=True)
        e = jnp.exp(A)
        t = e / jnp.sum(e, axis=0, keepdims=True)
        om = jnp.sum(rows * t, axis=0, keepdims=True)
        out_mil[b, :] = om[0]
        logits[b, :] = jnp.sum(om * mil_wT[...], axis=1) + mil_b[0]


def _slide_call(slide, p):
    fp = jax.ShapeDtypeStruct
    return pl.pallas_call(
        _slide_body,
        out_shape=[fp((B, FEAT), jnp.float32), fp((B, NUM_CLASSES), jnp.float32)],
    )(slide, p['laV_w'], p['laV_b'].reshape(1, -1),
      p['laU_w'], p['laU_b'].reshape(1, -1),
      p['laW_w'].reshape(1, -1), p['laW_b'].reshape(1, 1),
      p['mil_w'].T, p['mil_b'].reshape(1, -1))


RB = 256


def _z_body(xc, w, bias, zn):
    z = _lk(_dot(xc[...], w[...]) + bias[...], 0.01)
    n = jnp.sqrt(jnp.sum(z * z, axis=1, keepdims=True))
    zn[...] = z / (n + 1e-12)


def _z_call(xc, p):
    return pl.pallas_call(
        _z_body,
        grid=(BUF // RB,),
        in_specs=[pl.BlockSpec((RB, FEAT), lambda i: (i, 0)),
                  pl.BlockSpec((FEAT, HALF), lambda i: (0, 0)),
                  pl.BlockSpec((1, HALF), lambda i: (0, 0))],
        out_specs=pl.BlockSpec((RB, HALF), lambda i: (i, 0)),
        out_shape=jax.ShapeDtypeStruct((BUF, HALF), jnp.float32),
    )(xc, p['dsl_w'], p['dsl_b'].reshape(1, -1))


def _knn_body(znb, zn, nbr):
    sim = _dot_nt(znb[...], zn[...])
    iota = jax.lax.broadcasted_iota(jnp.int32, (RB, BUF), 1)
    cols = []
    for _ in range(K_NBR):
        m = jnp.max(sim, axis=1, keepdims=True)
        idxm = jnp.where(sim == m, iota, jnp.int32(2 ** 30))
        sel = jnp.min(idxm, axis=1, keepdims=True)
        cols.append(sel)
        sim = jnp.where(iota == sel, NEG, sim)
    nbr[...] = jnp.concatenate(cols, axis=1)


def _knn_call(zn):
    return pl.pallas_call(
        _knn_body,
        grid=(BUF // RB,),
        in_specs=[pl.BlockSpec((RB, HALF), lambda i: (i, 0)),
                  pl.BlockSpec((BUF, HALF), lambda i: (0, 0))],
        out_specs=pl.BlockSpec((RB, K_NBR), lambda i: (i, 0)),
        out_shape=jax.ShapeDtypeStruct((BUF, K_NBR), jnp.int32),
    )(zn, zn)


def _onehot(nbrb):
    iota = jax.lax.broadcasted_iota(jnp.int32, (RB, BUF), 1)
    oh = jnp.zeros((RB, BUF), jnp.float32)
    for j in range(K_NBR):
        oh += (nbrb[:, j:j + 1] == iota).astype(jnp.float32)
    return oh


def _eattr_body(nbrb, xc, eattr, dc):
    oh = _onehot(nbrb[...])
    eattr[...] = _dot(oh, xc[...]) * jnp.float32(1.0 / K_NBR)
    @pl.when(pl.program_id(0) == 0)
    def _():
        dc[...] = jnp.zeros_like(dc)
    dc[...] += jnp.sum(oh, axis=0, keepdims=True)


def _eattr_call(nbr, xc):
    fp = jax.ShapeDtypeStruct
    return pl.pallas_call(
        _eattr_body,
        grid=(BUF // RB,),
        in_specs=[pl.BlockSpec((RB, K_NBR), lambda i: (i, 0)),
                  pl.BlockSpec((BUF, FEAT), lambda i: (0, 0))],
        out_specs=[pl.BlockSpec((RB, FEAT), lambda i: (i, 0)),
                   pl.BlockSpec((1, BUF), lambda i: (0, 0))],
        out_shape=[fp((BUF, FEAT), jnp.float32), fp((1, BUF), jnp.float32)],
    )(nbr, xc)


def _hc1_body(xin, ea, W, attA, attB, xt, s1, s2):
    xtb = _dot(xin[...], W[...])
    etb = _dot(ea[...], W[...])
    xt[...] = xtb
    s1[...] = jnp.sum(xtb * attA[...], axis=1, keepdims=True)
    s2[...] = jnp.sum(etb * attB[...], axis=1, keepdims=True)


def _hc1_call(xin, ea, W, att):
    fp = jax.ShapeDtypeStruct
    return pl.pallas_call(
        _hc1_body,
        grid=(BUF // RB,),
        in_specs=[pl.BlockSpec((RB, FEAT), lambda i: (i, 0)),
                  pl.BlockSpec((RB, FEAT), lambda i: (i, 0)),
                  pl.BlockSpec((FEAT, FEAT), lambda i: (0, 0)),
                  pl.BlockSpec((1, FEAT), lambda i: (0, 0)),
                  pl.BlockSpec((1, FEAT), lambda i: (0, 0))],
        out_specs=[pl.BlockSpec((RB, FEAT), lambda i: (i, 0)),
                   pl.BlockSpec((RB, 1), lambda i: (i, 0)),
                   pl.BlockSpec((RB, 1), lambda i: (i, 0))],
        out_shape=[fp((BUF, FEAT), jnp.float32), fp((BUF, 1), jnp.float32),
                   fp((BUF, 1), jnp.float32)],
    )(xin, ea, W, att[:FEAT].reshape(1, -1), att[FEAT:].reshape(1, -1))


def _edge_a(nbrb, s1row, s2col):
    araw = _lk(s1row[...] + s2col[...], 0.2)
    return jnp.exp(araw) * _onehot(nbrb[...])


def _hc2_body(nbrb, s1row, s2col, asum):
    a = _edge_a(nbrb, s1row, s2col)
    @pl.when(pl.program_id(0) == 0)
    def _():
        asum[...] = jnp.zeros_like(asum)
    asum[...] += jnp.sum(a, axis=0, keepdims=True)


def _hc2_call(nbr, s1row, s2):
    return pl.pallas_call(
        _hc2_body,
        grid=(BUF // RB,),
        in_specs=[pl.BlockSpec((RB, K_NBR), lambda i: (i, 0)),
                  pl.BlockSpec((1, BUF), lambda i: (0, 0)),
                  pl.BlockSpec((RB, 1), lambda i: (i, 0))],
        out_specs=pl.BlockSpec((1, BUF), lambda i: (0, 0)),
        out_shape=jax.ShapeDtypeStruct((1, BUF), jnp.float32),
    )(nbr, s1row, s2)


def _hc3_body(nbrb, s1row, s2col, asum, xt, outv):
    a = _edge_a(nbrb, s1row, s2col)
    alpha = a / (asum[...] + 1e-16)
    oute = _dot(alpha, xt[...]) * jnp.float32(1.0 / K_NBR)
    @pl.when(pl.program_id(0) == 0)
    def _():
        outv[...] = jnp.zeros_like(outv)
    outv[...] += _dot_tn(alpha, oute)


def _hc3_call(nbr, s1row, s2, asum, xt):
    return pl.pallas_call(
        _hc3_body,
        grid=(BUF // RB,),
        in_specs=[pl.BlockSpec((RB, K_NBR), lambda i: (i, 0)),
                  pl.BlockSpec((1, BUF), lambda i: (0, 0)),
                  pl.BlockSpec((RB, 1), lambda i: (i, 0)),
                  pl.BlockSpec((1, BUF), lambda i: (0, 0)),
                  pl.BlockSpec((BUF, FEAT), lambda i: (0, 0))],
        out_specs=pl.BlockSpec((BUF, FEAT), lambda i: (0, 0)),
        out_shape=jax.ShapeDtypeStruct((BUF, FEAT), jnp.float32),
    )(nbr, s1row, s2, asum, xt)


def _gn_body(outv, dccol, hb, nw, nb, nms, gw, gb, h_out, o_out):
    dinv = jnp.where(dccol[...] > 0, 1.0 / dccol[...], 0.0)
    hpre = dinv * outv[...] + hb[...]
    mean = jnp.mean(hpre, axis=0, keepdims=True)
    cent = hpre - mean * nms[...]
    var = jnp.mean(cent * cent, axis=0, keepdims=True)
    h = nw[...] * cent / jnp.sqrt(var + 1e-5) + nb[...]
    h = _lk(h, 0.01)
    h_out[...] = h
    o_out[...] = _lk(_dot(h, gw[...]) + gb[...], 0.01)


def _gn_call(outv, dccol, p, ln):
    fp = jax.ShapeDtypeStruct
    return pl.pallas_call(
        _gn_body,
        out_shape=[fp((BUF, FEAT), jnp.float32), fp((BUF, HALF), jnp.float32)],
    )(outv, dccol, p[f'hgc{ln}_b'].reshape(1, -1),
      p[f'n{ln}_w'].reshape(1, -1), p[f'n{ln}_b'].reshape(1, -1),
      p[f'n{ln}_ms'].reshape(1, -1),
      p[f'gfc{ln}_w'], p[f'gfc{ln}_b'].reshape(1, -1))


def _hyperconv(xin, ea, nbr, dccol, p, ln):
    xt, s1, s2 = _hc1_call(xin, ea, p[f'hgc{ln}_w'], p[f'hgc{ln}_att'])
    s1row = s1.reshape(1, BUF)
    asum = _hc2_call(nbr, s1row, s2)
    outv = _hc3_call(nbr, s1row, s2, asum, xt)
    return _gn_call(outv, dccol, p, ln)


CB = 512


def _ga_body(out_full, ga1, ga1b, ga2row, t2):
    t1 = _dot_tn(out_full[...], ga1[...]) + ga1b[...]
    r = jnp.maximum(t1, 0.0)
    @pl.when(pl.program_id(0) == 0)
    def _():
        t2[...] = jnp.zeros_like(t2)
    t2[...] += jnp.sum(r * ga2row[...], axis=1, keepdims=True)


def _ga_call(out_full, p):
    return pl.pallas_call(
        _ga_body,
        grid=(BUF // CB,),
        in_specs=[pl.BlockSpec((BUF, 2 * FEAT), lambda i: (0, 0)),
                  pl.BlockSpec((BUF, CB), lambda i: (0, i)),
                  pl.BlockSpec((1, CB), lambda i: (0, i)),
                  pl.BlockSpec((1, CB), lambda i: (0, i))],
        out_specs=pl.BlockSpec((2 * FEAT, 1), lambda i: (0, 0)),
        out_shape=jax.ShapeDtypeStruct((2 * FEAT, 1), jnp.float32),
    )(out_full, p['ga1_w'], p['ga1_b'].reshape(1, -1),
      p['ga2_w'].reshape(1, -1))


def _fuse_body(a_in, ga2b, out4, gpw, gpb, om, milpw, milpb, fwT, fb, fl):
    a = jax.nn.sigmoid(a_in[...] + ga2b[0, 0])
    a = a - jnp.mean(a)
    og = out4[...] * a
    fusion = (_dot(og, gpw[...]) + gpb[...] +
              _dot(om[...], milpw[...]) + milpb[...])
    fl[...] = jnp.concatenate(
        [jnp.sum(fusion * fwT[c:c + 1, :], axis=1, keepdims=True)
         for c in range(NUM_CLASSES)], axis=1) + fb[...]


def _fuse_call(a_row, out4, out_mil, p):
    return pl.pallas_call(
        _fuse_body,
        out_shape=jax.ShapeDtypeStruct((B, NUM_CLASSES), jnp.float32),
    )(a_row, p['ga2_b'].reshape(1, 1), out4, p['gp_w'],
      p['gp_b'].reshape(1, -1), out_mil, p['milp_w'],
      p['milp_b'].reshape(1, -1), p['fus_w'].T, p['fus_b'].reshape(1, -1))


def kernel(x, params):
    p = params
    perm = jnp.asarray(_PERM, dtype=jnp.int32)
    xg = jnp.take(x, perm, axis=1).reshape(B, NG, N_INST // NG, FEAT)

    pseudo, subs = _mil_call(xg, p)
    out_mil, logits = _slide_call(pseudo.reshape(B, NG, FEAT), p)
    subs = subs.reshape(NG * B, NUM_CLASSES)

    xc = jnp.concatenate([out_mil, p['rehearsal']], axis=0)[:BUF]
    zn = _z_call(xc, p)
    nbr = _knn_call(zn)
    eattr, dc = _eattr_call(nbr, xc)
    dccol = dc.reshape(BUF, 1)

    h1, o1 = _hyperconv(xc, eattr, nbr, dccol, p, 1)
    h2, o2 = _hyperconv(h1, eattr, nbr, dccol, p, 2)

    out_full = jnp.concatenate([xc, o1, o2], axis=1)
    t2 = _ga_call(out_full, p)
    a_row = t2.reshape(1, 2 * FEAT)
    fusion_logits = _fuse_call(a_row, out_full[:B], out_mil, p)

    return (logits, subs, fusion_logits)


# speedup vs baseline: 5.2732x; 1.3547x over previous
import numpy as np
import jax
import jax.numpy as jnp
from jax.experimental import pallas as pl

FEAT = 512
NUM_CLASSES = 2
K_NBR = 8
BUF = 4096
NG = 8
B = 4
N_INST = 4096
ATT_D = 128
HALF = 256
PREC = None
EXACT = jax.lax.Precision.HIGHEST
NEG = -1e30

_PERM = np.random.RandomState(0).permutation(N_INST)


def _lk(x, s):
    return jnp.where(x >= 0, x, s * x)


def _b16(x):
    return x.astype(jnp.bfloat16).astype(jnp.float32)


def _rowdot(x, w):
    return jnp.sum(_b16(x) * _b16(w), axis=1, keepdims=True)


def _dot(a, b, prec=PREC):
    return jax.lax.dot_general(a, b, (((1,), (0,)), ((), ())), precision=prec,
                               preferred_element_type=jnp.float32)


def _dot_tn(a, b, prec=PREC):
    return jax.lax.dot_general(a, b, (((0,), (0,)), ((), ())), precision=prec,
                               preferred_element_type=jnp.float32)


def _dot_nt(a, b, prec=PREC):
    return jax.lax.dot_general(a, b, (((1,), (1,)), ((), ())), precision=prec,
                               preferred_element_type=jnp.float32)


def _mil_body(xg, dr_w, dr_b, aV_w, aV_b, aU_w, aU_b, aW, aW_b, sub_w, sub_b,
              pseudo, subs):
    xb = xg[0, 0]
    mid = jnp.maximum(_dot(xb, dr_w[...]) + dr_b[...], 0.0)
    V = jnp.tanh(_dot(mid, aV_w[...]) + aV_b[...])
    U = jax.nn.sigmoid(_dot(mid, aU_w[...]) + aU_b[...])
    A = _dot(V * U, aW[...]) + aW_b[0, 0]
    A = A - jnp.max(A, axis=0, keepdims=True)
    e = jnp.exp(A)
    t = e / jnp.sum(e, axis=0, keepdims=True)
    af = jnp.sum(mid * t, axis=0, keepdims=True)
    pseudo[0, 0, 0, :] = af[0]
    subs[0, 0, 0, :] = (_dot(af, sub_w[...]) + sub_b[...])[0]


def _mil_call(xg, p):
    fp = jax.ShapeDtypeStruct
    full = lambda shp: pl.BlockSpec(shp, lambda g, b: (0,) * len(shp))
    return pl.pallas_call(
        _mil_body,
        grid=(NG, B),
        in_specs=[
            pl.BlockSpec((1, 1, N_INST // NG, FEAT), lambda g, b: (b, g, 0, 0)),
            full((FEAT, FEAT)), full((1, FEAT)),
            full((FEAT, ATT_D)), full((1, ATT_D)),
            full((FEAT, ATT_D)), full((1, ATT_D)),
            full((ATT_D, 1)), full((1, 1)),
            full((FEAT, NUM_CLASSES)), full((1, NUM_CLASSES)),
        ],
        out_specs=[
            pl.BlockSpec((1, 1, 1, FEAT), lambda g, b: (b, g, 0, 0)),
            pl.BlockSpec((1, 1, 1, NUM_CLASSES), lambda g, b: (g, b, 0, 0)),
        ],
        out_shape=[fp((B, NG, 1, FEAT), jnp.float32),
                   fp((NG, B, 1, NUM_CLASSES), jnp.float32)],
    )(xg, p['dr_w'], p['dr_b'].reshape(1, -1),
      p['aV_w'], p['aV_b'].reshape(1, -1),
      p['aU_w'], p['aU_b'].reshape(1, -1),
      p['aW_w'], p['aW_b'].reshape(1, 1),
      p['sub_w'], p['sub_b'].reshape(1, -1))


def _slide_body(slide, laV_w, laV_b, laU_w, laU_b, laW, laW_b, mil_w, mil_b,
                out_mil, logits):
    oms = []
    for b in range(B):
        rows = slide[b]
        V = jnp.tanh(_dot(rows, laV_w[...]) + laV_b[...])
        U = jax.nn.sigmoid(_dot(rows, laU_w[...]) + laU_b[...])
        A = _dot(V * U, laW[...]) + laW_b[0, 0]
        A = A - jnp.max(A, axis=0, keepdims=True)
        e = jnp.exp(A)
        t = e / jnp.sum(e, axis=0, keepdims=True)
        om = _dot_tn(t, rows)
        out_mil[b, :] = om[0]
        oms.append(om)
    omall = jnp.concatenate(oms, axis=0)
    logits[...] = _dot(omall, mil_w[...]) + mil_b[...]


def _slide_call(slide, p):
    fp = jax.ShapeDtypeStruct
    return pl.pallas_call(
        _slide_body,
        out_shape=[fp((B, FEAT), jnp.float32), fp((B, NUM_CLASSES), jnp.float32)],
    )(slide, p['laV_w'], p['laV_b'].reshape(1, -1),
      p['laU_w'], p['laU_b'].reshape(1, -1),
      p['laW_w'], p['laW_b'].reshape(1, 1),
      p['mil_w'], p['mil_b'].reshape(1, -1))


RB = 256


def _z_body(xc, w, bias, zn):
    z = _lk(_dot(xc[...], w[...], prec=None) + bias[...], 0.01)
    n = jnp.sqrt(jnp.sum(z * z, axis=1, keepdims=True))
    zn[...] = z / (n + 1e-12)


def _z_call(xc, p):
    return pl.pallas_call(
        _z_body,
        grid=(BUF // RB,),
        in_specs=[pl.BlockSpec((RB, FEAT), lambda i: (i, 0)),
                  pl.BlockSpec((FEAT, HALF), lambda i: (0, 0)),
                  pl.BlockSpec((1, HALF), lambda i: (0, 0))],
        out_specs=pl.BlockSpec((RB, HALF), lambda i: (i, 0)),
        out_shape=jax.ShapeDtypeStruct((BUF, HALF), jnp.float32),
    )(xc, p['dsl_w'], p['dsl_b'].reshape(1, -1))


def _knn_body(znb, zn, nbr):
    sim = _dot_nt(znb[...], zn[...], prec=None)
    iota = jax.lax.broadcasted_iota(jnp.int32, (RB, BUF), 1)
    cols = []
    for _ in range(K_NBR):
        m = jnp.max(sim, axis=1, keepdims=True)
        idxm = jnp.where(sim == m, iota, jnp.int32(2 ** 30))
        sel = jnp.min(idxm, axis=1, keepdims=True)
        cols.append(sel)
        sim = jnp.where(iota == sel, NEG, sim)
    nbr[...] = jnp.concatenate(cols, axis=1)


def _knn_call(zn):
    return pl.pallas_call(
        _knn_body,
        grid=(BUF // RB,),
        in_specs=[pl.BlockSpec((RB, HALF), lambda i: (i, 0)),
                  pl.BlockSpec((BUF, HALF), lambda i: (0, 0))],
        out_specs=pl.BlockSpec((RB, K_NBR), lambda i: (i, 0)),
        out_shape=jax.ShapeDtypeStruct((BUF, K_NBR), jnp.int32),
    )(zn, zn)


def _onehot(nbrb):
    iota = jax.lax.broadcasted_iota(jnp.int32, (RB, BUF), 1)
    oh = jnp.zeros((RB, BUF), jnp.float32)
    for j in range(K_NBR):
        oh += (nbrb[:, j:j + 1] == iota).astype(jnp.float32)
    return oh


def _hc1_body(xin, xc, W, attA, attBc, xt, s1, q):
    xtb = _dot(xin[...], W[...])
    xt[...] = xtb
    s1[...] = jnp.sum(xtb * attA[...], axis=1, keepdims=True)
    q[...] = _dot(xc[...], _dot(W[...], attBc[...], prec=EXACT), prec=EXACT)


def _hc1_call(xin, xc, W, att):
    fp = jax.ShapeDtypeStruct
    return pl.pallas_call(
        _hc1_body,
        grid=(BUF // RB,),
        in_specs=[pl.BlockSpec((RB, FEAT), lambda i: (i, 0)),
                  pl.BlockSpec((RB, FEAT), lambda i: (i, 0)),
                  pl.BlockSpec((FEAT, FEAT), lambda i: (0, 0)),
                  pl.BlockSpec((1, FEAT), lambda i: (0, 0)),
                  pl.BlockSpec((FEAT, 1), lambda i: (0, 0))],
        out_specs=[pl.BlockSpec((RB, FEAT), lambda i: (i, 0)),
                   pl.BlockSpec((RB, 1), lambda i: (i, 0)),
                   pl.BlockSpec((RB, 1), lambda i: (i, 0))],
        out_shape=[fp((BUF, FEAT), jnp.float32), fp((BUF, 1), jnp.float32),
                   fp((BUF, 1), jnp.float32)],
    )(xin, xc, W, att[:FEAT].reshape(1, -1), att[FEAT:].reshape(-1, 1))


def _edge_a(nbrb, s1row, s2col):
    araw = _lk(s1row[...] + s2col, 0.2)
    return jnp.exp(araw) * _onehot(nbrb[...])


def _hc2_body(nbrb, s1row, qrow, s2, asum, dc):
    oh = _onehot(nbrb[...])
    s2c = jnp.sum(oh * qrow[...], axis=1, keepdims=True) * jnp.float32(1.0 / K_NBR)
    s2[...] = s2c
    a = jnp.exp(_lk(s1row[...] + s2c, 0.2)) * oh
    @pl.when(pl.program_id(0) == 0)
    def _():
        asum[...] = jnp.zeros_like(asum)
        dc[...] = jnp.zeros_like(dc)
    asum[...] += jnp.sum(a, axis=0, keepdims=True)
    dc[...] += jnp.sum(oh, axis=0, keepdims=True)


def _hc2_call(nbr, s1row, qrow):
    fp = jax.ShapeDtypeStruct
    return pl.pallas_call(
        _hc2_body,
        grid=(BUF // RB,),
        in_specs=[pl.BlockSpec((RB, K_NBR), lambda i: (i, 0)),
                  pl.BlockSpec((1, BUF), lambda i: (0, 0)),
                  pl.BlockSpec((1, BUF), lambda i: (0, 0))],
        out_specs=[pl.BlockSpec((RB, 1), lambda i: (i, 0)),
                   pl.BlockSpec((1, BUF), lambda i: (0, 0)),
                   pl.BlockSpec((1, BUF), lambda i: (0, 0))],
        out_shape=[fp((BUF, 1), jnp.float32), fp((1, BUF), jnp.float32),
                   fp((1, BUF), jnp.float32)],
    )(nbr, s1row, qrow)


def _hc3_body(nbrb, s1row, s2col, asum, xt, outv):
    a = _edge_a(nbrb, s1row, s2col[...])
    alpha = a / (asum[...] + 1e-16)
    oute = _dot(alpha, xt[...], prec=EXACT) * jnp.float32(1.0 / K_NBR)
    @pl.when(pl.program_id(0) == 0)
    def _():
        outv[...] = jnp.zeros_like(outv)
    outv[...] += _dot_tn(alpha, oute, prec=EXACT)


def _hc3_call(nbr, s1row, s2, asum, xt):
    return pl.pallas_call(
        _hc3_body,
        grid=(BUF // RB,),
        in_specs=[pl.BlockSpec((RB, K_NBR), lambda i: (i, 0)),
                  pl.BlockSpec((1, BUF), lambda i: (0, 0)),
                  pl.BlockSpec((RB, 1), lambda i: (i, 0)),
                  pl.BlockSpec((1, BUF), lambda i: (0, 0)),
                  pl.BlockSpec((BUF, FEAT), lambda i: (0, 0))],
        out_specs=pl.BlockSpec((BUF, FEAT), lambda i: (0, 0)),
        out_shape=jax.ShapeDtypeStruct((BUF, FEAT), jnp.float32),
    )(nbr, s1row, s2, asum, xt)


def _gn_body(outv, dccol, hb, nw, nb, nms, gw, gb, h_out, o_out):
    dinv = jnp.where(dccol[...] > 0, 1.0 / dccol[...], 0.0)
    hpre = dinv * outv[...] + hb[...]
    mean = jnp.mean(hpre, axis=0, keepdims=True)
    cent = hpre - mean * nms[...]
    var = jnp.mean(cent * cent, axis=0, keepdims=True)
    h = nw[...] * cent / jnp.sqrt(var + 1e-5) + nb[...]
    h = _lk(h, 0.01)
    h_out[...] = h
    o_out[...] = _lk(_dot(h, gw[...]) + gb[...], 0.01)


def _gn_call(outv, dccol, p, ln):
    fp = jax.ShapeDtypeStruct
    return pl.pallas_call(
        _gn_body,
        out_shape=[fp((BUF, FEAT), jnp.float32), fp((BUF, HALF), jnp.float32)],
    )(outv, dccol, p[f'hgc{ln}_b'].reshape(1, -1),
      p[f'n{ln}_w'].reshape(1, -1), p[f'n{ln}_b'].reshape(1, -1),
      p[f'n{ln}_ms'].reshape(1, -1),
      p[f'gfc{ln}_w'], p[f'gfc{ln}_b'].reshape(1, -1))


def _hyperconv(xin, xc, nbr, p, ln):
    xt, s1, q = _hc1_call(xin, xc, p[f'hgc{ln}_w'], p[f'hgc{ln}_att'])
    s1row = s1.reshape(1, BUF)
    s2, asum, dc = _hc2_call(nbr, s1row, q.reshape(1, BUF))
    outv = _hc3_call(nbr, s1row, s2, asum, xt)
    return _gn_call(outv, dc.reshape(BUF, 1), p, ln)


CB = 512


def _ga_body(out_full, ga1, ga1b, ga2, t2):
    t1 = _dot_tn(out_full[...], ga1[...]) + ga1b[...]
    r = jnp.maximum(t1, 0.0)
    @pl.when(pl.program_id(0) == 0)
    def _():
        t2[...] = jnp.zeros_like(t2)
    t2[...] += _dot(r, ga2[...])


def _ga_call(out_full, p):
    return pl.pallas_call(
        _ga_body,
        grid=(BUF // CB,),
        in_specs=[pl.BlockSpec((BUF, 2 * FEAT), lambda i: (0, 0)),
                  pl.BlockSpec((BUF, CB), lambda i: (0, i)),
                  pl.BlockSpec((1, CB), lambda i: (0, i)),
                  pl.BlockSpec((CB, 1), lambda i: (i, 0))],
        out_specs=pl.BlockSpec((2 * FEAT, 1), lambda i: (0, 0)),
        out_shape=jax.ShapeDtypeStruct((2 * FEAT, 1), jnp.float32),
    )(out_full, p['ga1_w'], p['ga1_b'].reshape(1, -1), p['ga2_w'])


def _fuse_body(a_in, ga2b, out4, gpw, gpb, om, milpw, milpb, fw, fb, fl):
    a = jax.nn.sigmoid(a_in[...] + ga2b[0, 0])
    a = a - jnp.mean(a)
    og = out4[...] * a
    fusion = (_dot(og, gpw[...]) + gpb[...] +
              _dot(om[...], milpw[...]) + milpb[...])
    fl[...] = _dot(fusion, fw[...]) + fb[...]


def _fuse_call(a_row, out4, out_mil, p):
    return pl.pallas_call(
        _fuse_body,
        out_shape=jax.ShapeDtypeStruct((B, NUM_CLASSES), jnp.float32),
    )(a_row, p['ga2_b'].reshape(1, 1), out4, p['gp_w'],
      p['gp_b'].reshape(1, -1), out_mil, p['milp_w'],
      p['milp_b'].reshape(1, -1), p['fus_w'], p['fus_b'].reshape(1, -1))


def kernel(x, params):
    p = params
    perm = jnp.asarray(_PERM, dtype=jnp.int32)
    xg = jnp.take(x, perm, axis=1).reshape(B, NG, N_INST // NG, FEAT)

    pseudo, subs = _mil_call(xg, p)
    out_mil, logits = _slide_call(pseudo.reshape(B, NG, FEAT), p)
    subs = subs.reshape(NG * B, NUM_CLASSES)

    xc = jnp.concatenate([out_mil, p['rehearsal']], axis=0)[:BUF]
    zn = _z_call(xc, p)
    nbr = _knn_call(zn)

    h1, o1 = _hyperconv(xc, xc, nbr, p, 1)
    h2, o2 = _hyperconv(h1, xc, nbr, p, 2)

    out_full = jnp.concatenate([xc, o1, o2], axis=1)
    t2 = _ga_call(out_full, p)
    a_row = t2.reshape(1, 2 * FEAT)
    fusion_logits = _fuse_call(a_row, out_full[:B], out_mil, p)

    return (logits, subs, fusion_logits)
